# Initial kernel scaffold; baseline (speedup 1.0000x reference)
import jax
import jax.numpy as jnp
from jax.experimental import pallas as pl


def kernel(x, edge_index, edge_attr, We1, be1, We2, be2, Wn1, bn1, Wn2, bn2):
    raise NotImplementedError("write your pallas kernel here")


# speedup vs baseline: 1.0036x; 1.0036x over previous
import jax
import jax.numpy as jnp
from jax import lax
from jax.experimental import pallas as pl
from jax.experimental.pallas import tpu as pltpu
from jax.experimental.pallas import tpu_sc as plsc

N = 10000
E = 160000
D = 256
DE = 16
H = 256
NODE_OUT = 5
EDGE_OUT = 2

NC, NS, L = 2, 16, 16
NW = NC * NS
EPW = E // NW
CH = 100
NCH = EPW // CH
NP = 4
HP = H // NP
KC = HP // L
N_PAD = 10240
SRS = N_PAD // NS
ZCH = 80


def _tables_body(x_ref, w_ref, o_ref):
    o_ref[0] = jnp.dot(x_ref[...], w_ref[0],
                       preferred_element_type=jnp.float32)


def _make_tables(x, wcat):
    bn = 2000
    return pl.pallas_call(
        _tables_body,
        grid=(N // bn, 2 * NP),
        in_specs=[
            pl.BlockSpec((bn, D), lambda i, j: (i, 0)),
            pl.BlockSpec((1, D, HP), lambda i, j: (j, 0, 0)),
        ],
        out_specs=pl.BlockSpec((1, bn, HP), lambda i, j: (j, i, 0)),
        out_shape=jax.ShapeDtypeStruct((2 * NP, N, HP), jnp.float32),
    )(x, wcat)


def _ec_body(ea_ref, w_ref, b_ref, o_ref):
    o_ref[0] = (jnp.dot(ea_ref[...], w_ref[0],
                        preferred_element_type=jnp.float32) + b_ref[0])


def _make_ec(ea, wc, be1r):
    be = 4000
    return pl.pallas_call(
        _ec_body,
        grid=(E // be, NP),
        in_specs=[
            pl.BlockSpec((be, DE), lambda i, j: (i, 0)),
            pl.BlockSpec((1, DE, HP), lambda i, j: (j, 0, 0)),
            pl.BlockSpec((1, 1, HP), lambda i, j: (j, 0, 0)),
        ],
        out_specs=pl.BlockSpec((1, be, HP), lambda i, j: (j, i, 0)),
        out_shape=jax.ShapeDtypeStruct((NP, E, HP), jnp.float32),
    )(ea, wc, be1r)


def _edge_sc(*refs):
    tabs_a = refs[0:NP]
    tabs_b = refs[NP:2 * NP]
    ecs = refs[2 * NP:3 * NP]
    src2d, dst2d, w2t, be2p = refs[3 * NP:3 * NP + 4]
    ep_out, agg_out = refs[3 * NP + 4:3 * NP + 6]
    (src_v, dst_v, a_buf, b_buf, ec_buf, h_buf, ep_buf, w2_v, be2_v,
     agg_sp, sem_a, sem_b, sem_c) = refs[3 * NP + 6:]

    c = lax.axis_index("c")
    s = lax.axis_index("s")
    wid = s * NC + c

    pltpu.sync_copy(src2d.at[wid], src_v)
    pltpu.sync_copy(dst2d.at[wid], dst_v)
    pltpu.sync_copy(w2t, w2_v)
    pltpu.sync_copy(be2p, be2_v)

    be2vec = be2_v[...]
    lanes = lax.iota(jnp.int32, L)
    mask15 = lanes == (L - 1)
    zeros = jnp.zeros((L,), jnp.float32)
    init0 = jnp.where(lanes == 0, be2vec, zeros)
    init1 = jnp.where(lanes == 1, be2vec, zeros)

    for p in range(NP):
        def _zero_h(i, carry):
            for k in range(KC):
                h_buf[i, pl.ds(k * L, L)] = zeros
            return carry
        lax.fori_loop(0, CH, _zero_h, 0)
        r0 = s * SRS
        for q in range(SRS // ZCH):
            pltpu.sync_copy(h_buf.at[pl.ds(0, ZCH)],
                            agg_sp.at[pl.ds(r0 + q * ZCH, ZCH)])
        plsc.subcore_barrier()

        wa = [w2_v[p, 0, pl.ds(k * L, L)] for k in range(KC)]
        wb = [w2_v[p, 1, pl.ds(k * L, L)] for k in range(KC)]

        def _chunk(ch, carry):
            cpa = pltpu.async_copy(tabs_a[p].at[src_v.at[ch]], a_buf, sem_a)
            cpb = pltpu.async_copy(tabs_b[p].at[dst_v.at[ch]], b_buf, sem_b)
            cpc = pltpu.async_copy(ecs[p].at[wid * NCH + ch], ec_buf, sem_c)
            cpa.wait()
            cpb.wait()
            cpc.wait()

            def _edge(e, inner):
                if p == 0:
                    acc0, acc1 = init0, init1
                else:
                    acc0, acc1 = zeros, zeros
                for k in range(KC):
                    sl = pl.ds(k * L, L)
                    h = jnp.maximum(a_buf[e, sl] + b_buf[e, sl] + ec_buf[e, sl],
                                    0.0)
                    h_buf[e, sl] = h
                    acc0 = acc0 + h * wa[k]
                    acc1 = acc1 + h * wb[k]
                r0v = plsc.cumsum(acc0)
                r1v = plsc.cumsum(acc1)
                i2 = (ch * CH + e) * EDGE_OUT
                idx0 = jnp.full((L,), 0, jnp.int32) + i2
                idx1 = idx0 + 1
                if p == 0:
                    plsc.store_scatter(ep_buf, [idx0], r0v, mask=mask15)
                    plsc.store_scatter(ep_buf, [idx1], r1v, mask=mask15)
                else:
                    old0 = plsc.load_gather(ep_buf, [idx0], mask=mask15)
                    old1 = plsc.load_gather(ep_buf, [idx1], mask=mask15)
                    plsc.store_scatter(ep_buf, [idx0], old0 + r0v, mask=mask15)
                    plsc.store_scatter(ep_buf, [idx1], old1 + r1v, mask=mask15)
                return inner
            lax.fori_loop(0, CH, _edge, 0)
            pltpu.sync_copy(h_buf, agg_sp.at[dst_v.at[ch]], add=True)
            return carry
        lax.fori_loop(0, NCH, _chunk, 0)
        plsc.subcore_barrier()
        pltpu.sync_copy(agg_sp.at[pl.ds(s * SRS, SRS)],
                        agg_out.at[c, p, pl.ds(s * SRS, SRS)])
        plsc.subcore_barrier()

    pltpu.sync_copy(ep_buf, ep_out.at[pl.ds(wid * EPW * EDGE_OUT,
                                            EPW * EDGE_OUT)])


def _make_edge_call():
    mesh = plsc.VectorSubcoreMesh(core_axis_name="c", subcore_axis_name="s",
                                  num_cores=NC, num_subcores=NS)
    return pl.kernel(
        _edge_sc,
        out_type=[
            jax.ShapeDtypeStruct((E * EDGE_OUT,), jnp.float32),
            jax.ShapeDtypeStruct((NC, NP, N_PAD, HP), jnp.float32),
        ],
        mesh=mesh,
        compiler_params=pltpu.CompilerParams(needs_layout_passes=False,
                                             use_tc_tiling_on_sc=False),
        scratch_types=[
            pltpu.VMEM((NCH, CH), jnp.int32),
            pltpu.VMEM((NCH, CH), jnp.int32),
            pltpu.VMEM((CH, HP), jnp.float32),
            pltpu.VMEM((CH, HP), jnp.float32),
            pltpu.VMEM((CH, HP), jnp.float32),
            pltpu.VMEM((CH, HP), jnp.float32),
            pltpu.VMEM((EPW * EDGE_OUT,), jnp.float32),
            pltpu.VMEM((NP, EDGE_OUT, HP), jnp.float32),
            pltpu.VMEM((L,), jnp.float32),
            pltpu.VMEM_SHARED((N_PAD, HP), jnp.float32),
            pltpu.SemaphoreType.DMA,
            pltpu.SemaphoreType.DMA,
            pltpu.SemaphoreType.DMA,
        ],
    )


def _node_body(x_ref, p_ref, wx_ref, wa_ref, b1_ref, w2_ref, b2_ref, o_ref):
    h = jnp.dot(x_ref[...], wx_ref[...], preferred_element_type=jnp.float32)
    for q in range(NP):
        agg_q = p_ref[0, q] + p_ref[1, q]
        h = h + jnp.dot(agg_q, wa_ref[q], preferred_element_type=jnp.float32)
    h = jnp.maximum(h + b1_ref[...], 0.0)
    o_ref[...] = (jnp.dot(h, w2_ref[...], preferred_element_type=jnp.float32)
                  + b2_ref[...])


def _node_call(x, aggp, wx, wa, b1r, w2, b2r):
    bn = 2000
    return pl.pallas_call(
        _node_body,
        grid=(N // bn,),
        in_specs=[
            pl.BlockSpec((bn, D), lambda i: (i, 0)),
            pl.BlockSpec((NC, NP, bn, HP), lambda i: (0, 0, i, 0)),
            pl.BlockSpec((D, H), lambda i: (0, 0)),
            pl.BlockSpec((NP, HP, H), lambda i: (0, 0, 0)),
            pl.BlockSpec((1, H), lambda i: (0, 0)),
            pl.BlockSpec((H, NODE_OUT), lambda i: (0, 0)),
            pl.BlockSpec((1, NODE_OUT), lambda i: (0, 0)),
        ],
        out_specs=pl.BlockSpec((bn, NODE_OUT), lambda i: (i, 0)),
        out_shape=jax.ShapeDtypeStruct((N, NODE_OUT), jnp.float32),
    )(x, aggp, wx, wa, b1r, w2, b2r)


_edge_call = _make_edge_call()


def kernel(x, edge_index, edge_attr, We1, be1, We2, be2, Wn1, bn1, Wn2, bn2):
    wcat = jnp.concatenate([We1[:D], We1[D:2 * D]], axis=1)
    wcat_r = wcat.reshape(D, 2 * NP, HP).transpose(1, 0, 2)
    tabs = _make_tables(x, wcat_r)
    wc_r = We1[2 * D:].reshape(DE, NP, HP).transpose(1, 0, 2)
    ec = _make_ec(edge_attr, wc_r, be1.reshape(NP, 1, HP))

    src2d = edge_index[0].reshape(NW, NCH, CH)
    dst2d = edge_index[1].reshape(NW, NCH, CH)
    w2t = We2.reshape(NP, HP, EDGE_OUT).transpose(0, 2, 1)
    be2p = jnp.zeros((L,), jnp.float32).at[:EDGE_OUT].set(be2)

    args = ([tabs[j] for j in range(2 * NP)]
            + [ec[j].reshape(E // CH, CH, HP) for j in range(NP)]
            + [src2d, dst2d, w2t, be2p])
    ep, aggp = _edge_call(*args)

    node_pred = _node_call(x, aggp, Wn1[:D],
                           Wn1[D:].reshape(NP, HP, H), bn1.reshape(1, H),
                           Wn2, bn2.reshape(1, NODE_OUT))
    return node_pred, ep.reshape(E, EDGE_OUT)


# speedup vs baseline: 1.2020x; 1.1977x over previous
import jax
import jax.numpy as jnp
from jax import lax
from jax.experimental import pallas as pl
from jax.experimental.pallas import tpu as pltpu
from jax.experimental.pallas import tpu_sc as plsc

N = 10000
E = 160000
D = 256
DE = 16
H = 256
NODE_OUT = 5
EDGE_OUT = 2

NC, NS, L = 2, 16, 16
NW = NC * NS
EPW = E // NW
CH = 100
NCH = EPW // CH
NP = 4
HP = H // NP
KC = HP // L
N_PAD = 10240
SRS = N_PAD // NS
ZCH = 80


def _tables_body(x_ref, w_ref, o_ref):
    o_ref[0] = jnp.dot(x_ref[...], w_ref[0],
                       preferred_element_type=jnp.float32)


def _make_tables(x, wcat):
    bn = 2000
    return pl.pallas_call(
        _tables_body,
        grid=(N // bn, 2 * NP),
        in_specs=[
            pl.BlockSpec((bn, D), lambda i, j: (i, 0)),
            pl.BlockSpec((1, D, HP), lambda i, j: (j, 0, 0)),
        ],
        out_specs=pl.BlockSpec((1, bn, HP), lambda i, j: (j, i, 0)),
        out_shape=jax.ShapeDtypeStruct((2 * NP, N, HP), jnp.float32),
    )(x, wcat)


def _ec_body(ea_ref, w_ref, b_ref, o_ref):
    y = (jnp.dot(ea_ref[...], w_ref[0],
                 preferred_element_type=jnp.float32) + b_ref[0])
    o_ref[0] = y.reshape(o_ref.shape[1:])


def _make_ec(ea, wc, be1r):
    be = 4000
    return pl.pallas_call(
        _ec_body,
        grid=(E // be, NP),
        in_specs=[
            pl.BlockSpec((be, DE), lambda i, j: (i, 0)),
            pl.BlockSpec((1, DE, HP), lambda i, j: (j, 0, 0)),
            pl.BlockSpec((1, 1, HP), lambda i, j: (j, 0, 0)),
        ],
        out_specs=pl.BlockSpec((1, be // CH, CH, HP), lambda i, j: (j, i, 0, 0)),
        out_shape=jax.ShapeDtypeStruct((NP, E // CH, CH, HP), jnp.float32),
    )(ea, wc, be1r)


def _edge_sc(tabs, ecs4, ei4, w2t, be2p,
             ep_out, agg_out,
             src_v, dst_v, a_buf, b_buf, ec_buf, h_buf, ep_buf, w2_v, be2_v,
             agg_sp, sem_a, sem_b, sem_c):

    c = lax.axis_index("c")
    s = lax.axis_index("s")
    wid = s * NC + c

    pltpu.sync_copy(ei4.at[0].at[wid], src_v)
    pltpu.sync_copy(ei4.at[1].at[wid], dst_v)
    pltpu.sync_copy(w2t, w2_v)
    pltpu.sync_copy(be2p, be2_v)

    be2vec = be2_v[...]
    lanes = lax.iota(jnp.int32, L)
    mask15 = lanes == (L - 1)
    zeros = jnp.zeros((L,), jnp.float32)
    init0 = jnp.where(lanes == 0, be2vec, zeros)
    init1 = jnp.where(lanes == 1, be2vec, zeros)

    for p in range(NP):
        def _zero_h(i, carry):
            for k in range(KC):
                h_buf[i, pl.ds(k * L, L)] = zeros
            return carry
        lax.fori_loop(0, CH, _zero_h, 0)
        r0 = s * SRS
        for q in range(SRS // ZCH):
            pltpu.sync_copy(h_buf.at[pl.ds(0, ZCH)],
                            agg_sp.at[pl.ds(r0 + q * ZCH, ZCH)])
        plsc.subcore_barrier()

        wa = [w2_v[p, 0, pl.ds(k * L, L)] for k in range(KC)]
        wb = [w2_v[p, 1, pl.ds(k * L, L)] for k in range(KC)]

        def _chunk(ch, carry):
            cpa = pltpu.async_copy(tabs.at[p].at[src_v.at[ch]], a_buf, sem_a)
            cpb = pltpu.async_copy(tabs.at[NP + p].at[dst_v.at[ch]], b_buf,
                                   sem_b)
            cpc = pltpu.async_copy(ecs4.at[p].at[wid * NCH + ch], ec_buf,
                                   sem_c)
            cpa.wait()
            cpb.wait()
            cpc.wait()

            def _edge(e, inner):
                if p == 0:
                    acc0, acc1 = init0, init1
                else:
                    acc0, acc1 = zeros, zeros
                for k in range(KC):
                    sl = pl.ds(k * L, L)
                    h = jnp.maximum(a_buf[e, sl] + b_buf[e, sl] + ec_buf[e, sl],
                                    0.0)
                    h_buf[e, sl] = h
                    acc0 = acc0 + h * wa[k]
                    acc1 = acc1 + h * wb[k]
                r0v = plsc.cumsum(acc0)
                r1v = plsc.cumsum(acc1)
                erow = jnp.full((L,), 0, jnp.int32) + (ch * CH + e)
                col0 = jnp.zeros((L,), jnp.int32)
                col1 = col0 + 1
                if p == 0:
                    plsc.store_scatter(ep_buf, [erow, col0], r0v, mask=mask15)
                    plsc.store_scatter(ep_buf, [erow, col1], r1v, mask=mask15)
                else:
                    old0 = plsc.load_gather(ep_buf, [erow, col0], mask=mask15)
                    old1 = plsc.load_gather(ep_buf, [erow, col1], mask=mask15)
                    plsc.store_scatter(ep_buf, [erow, col0], old0 + r0v,
                                       mask=mask15)
                    plsc.store_scatter(ep_buf, [erow, col1], old1 + r1v,
                                       mask=mask15)
                return inner
            lax.fori_loop(0, CH, _edge, 0)
            pltpu.sync_copy(h_buf, agg_sp.at[dst_v.at[ch]], add=True)
            return carry
        lax.fori_loop(0, NCH, _chunk, 0)
        plsc.subcore_barrier()
        pltpu.sync_copy(agg_sp.at[pl.ds(s * SRS, SRS)],
                        agg_out.at[c, p, pl.ds(s * SRS, SRS)])
        plsc.subcore_barrier()

    pltpu.sync_copy(ep_buf, ep_out.at[pl.ds(wid * EPW, EPW)])


def _make_edge_call():
    mesh = plsc.VectorSubcoreMesh(core_axis_name="c", subcore_axis_name="s",
                                  num_cores=NC, num_subcores=NS)
    return pl.kernel(
        _edge_sc,
        out_type=[
            jax.ShapeDtypeStruct((E, EDGE_OUT), jnp.float32),
            jax.ShapeDtypeStruct((NC, NP, N_PAD, HP), jnp.float32),
        ],
        mesh=mesh,
        compiler_params=pltpu.CompilerParams(needs_layout_passes=False,
                                             use_tc_tiling_on_sc=False),
        scratch_types=[
            pltpu.VMEM((NCH, CH), jnp.int32),
            pltpu.VMEM((NCH, CH), jnp.int32),
            pltpu.VMEM((CH, HP), jnp.float32),
            pltpu.VMEM((CH, HP), jnp.float32),
            pltpu.VMEM((CH, HP), jnp.float32),
            pltpu.VMEM((CH, HP), jnp.float32),
            pltpu.VMEM((EPW, EDGE_OUT), jnp.float32),
            pltpu.VMEM((NP, EDGE_OUT, HP), jnp.float32),
            pltpu.VMEM((L,), jnp.float32),
            pltpu.VMEM_SHARED((N_PAD, HP), jnp.float32),
            pltpu.SemaphoreType.DMA,
            pltpu.SemaphoreType.DMA,
            pltpu.SemaphoreType.DMA,
        ],
    )


def _node_body(x_ref, p_ref, wx_ref, wa_ref, b1_ref, w2_ref, b2_ref, o_ref):
    h = jnp.dot(x_ref[...], wx_ref[...], preferred_element_type=jnp.float32)
    for q in range(NP):
        agg_q = p_ref[0, q] + p_ref[1, q]
        h = h + jnp.dot(agg_q, wa_ref[q], preferred_element_type=jnp.float32)
    h = jnp.maximum(h + b1_ref[...], 0.0)
    o_ref[...] = (jnp.dot(h, w2_ref[...], preferred_element_type=jnp.float32)
                  + b2_ref[...])


def _node_call(x, aggp, wx, wa, b1r, w2, b2r):
    bn = 2000
    return pl.pallas_call(
        _node_body,
        grid=(N // bn,),
        in_specs=[
            pl.BlockSpec((bn, D), lambda i: (i, 0)),
            pl.BlockSpec((NC, NP, bn, HP), lambda i: (0, 0, i, 0)),
            pl.BlockSpec((D, H), lambda i: (0, 0)),
            pl.BlockSpec((NP, HP, H), lambda i: (0, 0, 0)),
            pl.BlockSpec((1, H), lambda i: (0, 0)),
            pl.BlockSpec((H, NODE_OUT), lambda i: (0, 0)),
            pl.BlockSpec((1, NODE_OUT), lambda i: (0, 0)),
        ],
        out_specs=pl.BlockSpec((bn, NODE_OUT), lambda i: (i, 0)),
        out_shape=jax.ShapeDtypeStruct((N, NODE_OUT), jnp.float32),
    )(x, aggp, wx, wa, b1r, w2, b2r)


_edge_call = _make_edge_call()


def kernel(x, edge_index, edge_attr, We1, be1, We2, be2, Wn1, bn1, Wn2, bn2):
    wcat = jnp.concatenate([We1[:D], We1[D:2 * D]], axis=1)
    wcat_r = wcat.reshape(D, 2 * NP, HP).transpose(1, 0, 2)
    tabs = _make_tables(x, wcat_r)
    wc_r = We1[2 * D:].reshape(DE, NP, HP).transpose(1, 0, 2)
    ec = _make_ec(edge_attr, wc_r, be1.reshape(NP, 1, HP))

    ei4 = edge_index.reshape(2, NW, NCH, CH)
    w2t = We2.reshape(NP, HP, EDGE_OUT).transpose(0, 2, 1)
    be2p = jnp.zeros((L,), jnp.float32).at[:EDGE_OUT].set(be2)

    ep, aggp = _edge_call(tabs, ec, ei4, w2t, be2p)

    node_pred = _node_call(x, aggp, Wn1[:D],
                           Wn1[D:].reshape(NP, HP, H), bn1.reshape(1, H),
                           Wn2, bn2.reshape(1, NODE_OUT))
    return node_pred, ep
